# baseline (speedup 1.0000x reference)
import functools

import jax
import jax.numpy as jnp
import numpy as np
from jax import lax
from jax.experimental import pallas as pl
from jax.experimental.pallas import tpu as pltpu
from jax.experimental.pallas import tpu_sc as plsc

N = 10000
NA = 10240
PADN = NA - 8
E = 320000
L = 3
_W = [float(np.exp(-l)) for l in range(L + 1)]

NC = 2
NS = 16
LANES = 16
C = 64
NCH = 320
NR = 4
EPT = C * NCH
EPAD = EPT * NS
NPT = NA // NS
RZ = 32


@functools.cache
def _make_prop3(h):
    mesh = plsc.VectorSubcoreMesh(core_axis_name="c", subcore_axis_name="s")
    out_sds = jax.ShapeDtypeStruct((NA, h), jnp.float32)

    @functools.partial(
        pl.kernel,
        out_type=(out_sds,) * 6,
        mesh=mesh,
        scratch_types=[
            [pltpu.VMEM((C,), jnp.int32) for _ in range(NR)],
            [pltpu.VMEM((C,), jnp.int32) for _ in range(NR)],
            [pltpu.VMEM((C, h), jnp.float32) for _ in range(NR)],
            pltpu.VMEM((RZ, h), jnp.float32),
            pltpu.VMEM_SHARED((NA, h), jnp.float32),
            [pltpu.SemaphoreType.DMA for _ in range(NR)],
            [pltpu.SemaphoreType.DMA for _ in range(NR)],
        ],
        compiler_params=pltpu.CompilerParams(use_tc_tiling_on_sc=False),
    )
    def prop3(tblA, tblB, src2, dst2, o1A, o1B, o2A, o2B, o3A, o3B,
              sib, dib, gb, zblk, acc, isem, gsem):
        c = lax.axis_index("c")
        s = lax.axis_index("s")

        per_row = h // LANES

        def zb(i, _):
            zblk[i // per_row, pl.ds((i % per_row) * LANES, LANES)] = (
                jnp.zeros((LANES,), jnp.float32))
            return 0

        lax.fori_loop(0, RZ * per_row, zb, 0)

        def idx_fetch(ch, b):
            row = s * NCH + ch
            pltpu.async_copy(src2.at[row], sib[b], isem[b])
            pltpu.async_copy(dst2.at[row], dib[b], isem[b])

        def idx_wait(ch, b):
            row = s * NCH + ch
            pltpu.make_async_copy(src2.at[row], sib[b], isem[b]).wait()
            pltpu.make_async_copy(dst2.at[row], dib[b], isem[b]).wait()

        def edge_phase(tref):
            for k in range(NR):
                idx_fetch(k, k)
            for k in range(NR - 1):
                idx_wait(k, k)
                pltpu.async_copy(tref.at[sib[k]], gb[k], gsem[k])

            def body(j, _):
                for b in range(NR):
                    i = j * NR + b
                    nb = (b + NR - 1) % NR

                    pltpu.make_async_copy(
                        tref.at[sib[b]], gb[b], gsem[b]).wait()

                    @pl.when(i + NR - 1 < NCH)
                    def _():
                        idx_wait(i + NR - 1, nb)
                        pltpu.async_copy(tref.at[sib[nb]], gb[nb], gsem[nb])

                    pltpu.sync_copy(gb[b], acc.at[dib[b]], add=True)

                    @pl.when(i + NR < NCH)
                    def _():
                        idx_fetch(i + NR, b)
                return 0

            lax.fori_loop(0, NCH // NR, body, 0)

        def one_round(ta, tb, oa, ob):
            for k in range(NPT // RZ):
                pltpu.sync_copy(zblk, acc.at[pl.ds(s * NPT + k * RZ, RZ)])
            plsc.subcore_barrier()

            @pl.when(c == 0)
            def _():
                edge_phase(ta)

            @pl.when(c == 1)
            def _():
                edge_phase(tb)

            plsc.subcore_barrier()

            @pl.when(c == 0)
            def _():
                pltpu.sync_copy(acc.at[pl.ds(s * NPT, NPT)],
                                oa.at[pl.ds(s * NPT, NPT)])

            @pl.when(c == 1)
            def _():
                pltpu.sync_copy(acc.at[pl.ds(s * NPT, NPT)],
                                ob.at[pl.ds(s * NPT, NPT)])

            plsc.subcore_barrier()

        one_round(tblA, tblB, o1A, o1B)
        one_round(o1A, o1B, o2A, o2B)
        one_round(o2A, o2B, o3A, o3B)

    return prop3


R = 1000
NB = N // R


def _prep_body(x_ref, c1_ref, c2_ref, c3_ref, dis_ref, v0a_ref, v0b_ref):
    deg = (_W[0] + _W[1] * c1_ref[...] + _W[2] * c2_ref[...]
           + _W[3] * c3_ref[...])
    safe = jnp.where(deg > 0, deg, 1.0)
    dis = jnp.where(deg > 0, 1.0 / jnp.sqrt(safe), 0.0)
    dis_ref[...] = dis
    xv = x_ref[...] * dis[:, :1]
    v0a_ref[...] = xv[:, :64]
    v0b_ref[...] = xv[:, 64:]


def _prep(x, c1, c2, c3):
    return pl.pallas_call(
        _prep_body,
        grid=(NB,),
        in_specs=[
            pl.BlockSpec((R, 128), lambda i: (i, 0)),
            pl.BlockSpec((R, 16), lambda i: (i, 0)),
            pl.BlockSpec((R, 16), lambda i: (i, 0)),
            pl.BlockSpec((R, 16), lambda i: (i, 0)),
        ],
        out_specs=[
            pl.BlockSpec((R, 16), lambda i: (i, 0)),
            pl.BlockSpec((R, 64), lambda i: (i, 0)),
            pl.BlockSpec((R, 64), lambda i: (i, 0)),
        ],
        out_shape=[
            jax.ShapeDtypeStruct((N, 16), jnp.float32),
            jax.ShapeDtypeStruct((NA, 64), jnp.float32),
            jax.ShapeDtypeStruct((NA, 64), jnp.float32),
        ],
    )(x, c1, c2, c3)


def _layer_body(scale_out, va_ref, vb_ref, o1a_ref, o1b_ref, o2a_ref,
                o2b_ref, o3a_ref, o3b_ref, dis_ref, w_ref, b_ref,
                outa_ref, outb_ref):
    def full(ra, rb):
        return jnp.concatenate([ra[...], rb[...]], axis=1)

    z = (_W[0] * full(va_ref, vb_ref) + _W[1] * full(o1a_ref, o1b_ref)
         + _W[2] * full(o2a_ref, o2b_ref) + _W[3] * full(o3a_ref, o3b_ref))
    d1 = dis_ref[...][:, :1]
    z = z * d1
    o = jnp.dot(z, w_ref[...], preferred_element_type=jnp.float32)
    o = jnp.maximum(o + b_ref[...], 0.0)
    if scale_out:
        o = o * d1
    outa_ref[...] = o[:, :128]
    outb_ref[...] = o[:, 128:]


def _layer(va, vb, o1a, o1b, o2a, o2b, o3a, o3b, dis, w, b, scale_out):
    h = va.shape[1]
    fin = 2 * h
    fspec = pl.BlockSpec((R, h), lambda i: (i, 0))
    return pl.pallas_call(
        functools.partial(_layer_body, scale_out),
        grid=(NB,),
        in_specs=[
            fspec, fspec, fspec, fspec, fspec, fspec, fspec, fspec,
            pl.BlockSpec((R, 16), lambda i: (i, 0)),
            pl.BlockSpec((fin, 256), lambda i: (0, 0)),
            pl.BlockSpec((1, 256), lambda i: (0, 0)),
        ],
        out_specs=[
            pl.BlockSpec((R, 128), lambda i: (i, 0)),
            pl.BlockSpec((R, 128), lambda i: (i, 0)),
        ],
        out_shape=[
            jax.ShapeDtypeStruct((NA, 128), jnp.float32),
            jax.ShapeDtypeStruct((NA, 128), jnp.float32),
        ],
    )(va, vb, o1a, o1b, o2a, o2b, o3a, o3b, dis, w, b)


def _final_body(ha_ref, hb_ref, w_ref, b_ref, out_ref):
    hfull = jnp.concatenate([ha_ref[...], hb_ref[...]], axis=1)
    o = jnp.dot(hfull, w_ref[...], preferred_element_type=jnp.float32)
    o = o + b_ref[...]
    m = jnp.max(o, axis=1, keepdims=True)
    e = jnp.exp(o - m)
    se = jnp.sum(e, axis=1, keepdims=True)
    out_ref[...] = o - m - jnp.log(se)


def _final(ha, hb, wl, bl):
    return pl.pallas_call(
        _final_body,
        grid=(NB,),
        in_specs=[
            pl.BlockSpec((R, 128), lambda i: (i, 0)),
            pl.BlockSpec((R, 128), lambda i: (i, 0)),
            pl.BlockSpec((256, 64), lambda i: (0, 0)),
            pl.BlockSpec((1, 64), lambda i: (0, 0)),
        ],
        out_specs=pl.BlockSpec((R, 64), lambda i: (i, 0)),
        out_shape=jax.ShapeDtypeStruct((N, 64), jnp.float32),
    )(ha, hb, wl, bl)


def kernel(x, edge_index, batch, W0, b0, W1, b1, W2, b2, Wl, bl):
    del batch
    src = edge_index[0]
    dst = edge_index[1]
    pad = jnp.full((EPAD - E,), PADN, jnp.int32)
    src2 = jnp.concatenate([src, pad]).reshape(NS * NCH, C)
    dst2 = jnp.concatenate([dst, pad]).reshape(NS * NCH, C)

    ones16 = jnp.ones((NA, 16), jnp.float32)
    c1, _, c2, _, c3, _ = _make_prop3(16)(ones16, ones16, src2, dst2)

    dis, v0a, v0b = _prep(x, c1[:N], c2[:N], c3[:N])

    o1a, o1b, o2a, o2b, o3a, o3b = _make_prop3(64)(v0a, v0b, src2, dst2)
    v1a, v1b = _layer(v0a, v0b, o1a, o1b, o2a, o2b, o3a, o3b, dis,
                      W0, b0.reshape(1, 256), True)

    o1a, o1b, o2a, o2b, o3a, o3b = _make_prop3(128)(v1a, v1b, src2, dst2)
    v2a, v2b = _layer(v1a, v1b, o1a, o1b, o2a, o2b, o3a, o3b, dis,
                      W1, b1.reshape(1, 256), True)

    o1a, o1b, o2a, o2b, o3a, o3b = _make_prop3(128)(v2a, v2b, src2, dst2)
    h3a, h3b = _layer(v2a, v2b, o1a, o1b, o2a, o2b, o3a, o3b, dis,
                      W2, b2.reshape(1, 256), False)

    return _final(h3a, h3b, Wl, bl.reshape(1, 64))


# speedup vs baseline: 1.0052x; 1.0052x over previous
import functools

import jax
import jax.numpy as jnp
import numpy as np
from jax import lax
from jax.experimental import pallas as pl
from jax.experimental.pallas import tpu as pltpu
from jax.experimental.pallas import tpu_sc as plsc

N = 10000
NA = 10240
PADN = NA - 8
E = 320000
L = 3
_W = [float(np.exp(-l)) for l in range(L + 1)]

NC = 2
NS = 16
LANES = 16
C = 128
NCH = 160
NI = 4
NG = 2
EPT = C * NCH
EPAD = EPT * NS
NPT = NA // NS
RZ = 32


@functools.cache
def _make_prop3(h):
    mesh = plsc.VectorSubcoreMesh(core_axis_name="c", subcore_axis_name="s")
    out_sds = jax.ShapeDtypeStruct((NA, h), jnp.float32)

    @functools.partial(
        pl.kernel,
        out_type=(out_sds,) * 6,
        mesh=mesh,
        scratch_types=[
            [pltpu.VMEM((C,), jnp.int32) for _ in range(NI)],
            [pltpu.VMEM((C,), jnp.int32) for _ in range(NI)],
            [pltpu.VMEM((C, h), jnp.float32) for _ in range(NG)],
            pltpu.VMEM((RZ, h), jnp.float32),
            pltpu.VMEM_SHARED((NA, h), jnp.float32),
            [pltpu.SemaphoreType.DMA for _ in range(NI)],
            [pltpu.SemaphoreType.DMA for _ in range(NG)],
            [pltpu.SemaphoreType.DMA for _ in range(NG)],
        ],
        compiler_params=pltpu.CompilerParams(use_tc_tiling_on_sc=False),
    )
    def prop3(tblA, tblB, src2, dst2, o1A, o1B, o2A, o2B, o3A, o3B,
              sib, dib, gb, zblk, acc, isem, gsem, ssem):
        c = lax.axis_index("c")
        s = lax.axis_index("s")

        per_row = h // LANES

        def zb(i, _):
            zblk[i // per_row, pl.ds((i % per_row) * LANES, LANES)] = (
                jnp.zeros((LANES,), jnp.float32))
            return 0

        lax.fori_loop(0, RZ * per_row, zb, 0)

        def idx_fetch(ch, b):
            row = s * NCH + ch
            pltpu.async_copy(src2.at[row], sib[b], isem[b])
            pltpu.async_copy(dst2.at[row], dib[b], isem[b])

        def idx_wait(ch, b):
            row = s * NCH + ch
            pltpu.make_async_copy(src2.at[row], sib[b], isem[b]).wait()
            pltpu.make_async_copy(dst2.at[row], dib[b], isem[b]).wait()

        def edge_phase(tref):
            for k in range(3):
                idx_fetch(k, k)
            idx_wait(0, 0)
            pltpu.async_copy(tref.at[sib[0]], gb[0], gsem[0])

            def body(j, _):
                for b4 in range(4):
                    i = j * 4 + b4
                    g = b4 % NG
                    ng = 1 - g
                    pg = (b4 + 3) % 4

                    pltpu.make_async_copy(
                        tref.at[sib[b4]], gb[g], gsem[g]).wait()

                    @pl.when(i > 0)
                    def _():
                        pltpu.make_async_copy(
                            gb[ng], acc.at[dib[pg]], ssem[ng]).wait()

                    @pl.when(i + 1 < NCH)
                    def _():
                        idx_wait(i + 1, (b4 + 1) % 4)
                        pltpu.async_copy(
                            tref.at[sib[(b4 + 1) % 4]], gb[ng], gsem[ng])

                    pltpu.async_copy(gb[g], acc.at[dib[b4]], ssem[g],
                                     add=True)

                    @pl.when(i + 3 < NCH)
                    def _():
                        idx_fetch(i + 3, pg)
                return 0

            lax.fori_loop(0, NCH // 4, body, 0)
            pltpu.make_async_copy(
                gb[(NCH - 1) % NG], acc.at[dib[(NCH - 1) % 4]],
                ssem[(NCH - 1) % NG]).wait()

        def one_round(ta, tb, oa, ob):
            for k in range(NPT // RZ):
                pltpu.sync_copy(zblk, acc.at[pl.ds(s * NPT + k * RZ, RZ)])
            plsc.subcore_barrier()

            @pl.when(c == 0)
            def _():
                edge_phase(ta)

            @pl.when(c == 1)
            def _():
                edge_phase(tb)

            plsc.subcore_barrier()

            @pl.when(c == 0)
            def _():
                pltpu.sync_copy(acc.at[pl.ds(s * NPT, NPT)],
                                oa.at[pl.ds(s * NPT, NPT)])

            @pl.when(c == 1)
            def _():
                pltpu.sync_copy(acc.at[pl.ds(s * NPT, NPT)],
                                ob.at[pl.ds(s * NPT, NPT)])

            plsc.subcore_barrier()

        one_round(tblA, tblB, o1A, o1B)
        one_round(o1A, o1B, o2A, o2B)
        one_round(o2A, o2B, o3A, o3B)

    return prop3


R = 1000
NB = N // R


def _prep_body(x_ref, c1_ref, c2_ref, c3_ref, dis_ref, v0a_ref, v0b_ref):
    deg = (_W[0] + _W[1] * c1_ref[...] + _W[2] * c2_ref[...]
           + _W[3] * c3_ref[...])
    safe = jnp.where(deg > 0, deg, 1.0)
    dis = jnp.where(deg > 0, 1.0 / jnp.sqrt(safe), 0.0)
    dis_ref[...] = dis
    xv = x_ref[...] * dis[:, :1]
    v0a_ref[...] = xv[:, :64]
    v0b_ref[...] = xv[:, 64:]


def _prep(x, c1, c2, c3):
    return pl.pallas_call(
        _prep_body,
        grid=(NB,),
        in_specs=[
            pl.BlockSpec((R, 128), lambda i: (i, 0)),
            pl.BlockSpec((R, 16), lambda i: (i, 0)),
            pl.BlockSpec((R, 16), lambda i: (i, 0)),
            pl.BlockSpec((R, 16), lambda i: (i, 0)),
        ],
        out_specs=[
            pl.BlockSpec((R, 16), lambda i: (i, 0)),
            pl.BlockSpec((R, 64), lambda i: (i, 0)),
            pl.BlockSpec((R, 64), lambda i: (i, 0)),
        ],
        out_shape=[
            jax.ShapeDtypeStruct((N, 16), jnp.float32),
            jax.ShapeDtypeStruct((NA, 64), jnp.float32),
            jax.ShapeDtypeStruct((NA, 64), jnp.float32),
        ],
    )(x, c1, c2, c3)


def _layer_body(scale_out, va_ref, vb_ref, o1a_ref, o1b_ref, o2a_ref,
                o2b_ref, o3a_ref, o3b_ref, dis_ref, w_ref, b_ref,
                outa_ref, outb_ref):
    def full(ra, rb):
        return jnp.concatenate([ra[...], rb[...]], axis=1)

    z = (_W[0] * full(va_ref, vb_ref) + _W[1] * full(o1a_ref, o1b_ref)
         + _W[2] * full(o2a_ref, o2b_ref) + _W[3] * full(o3a_ref, o3b_ref))
    d1 = dis_ref[...][:, :1]
    z = z * d1
    o = jnp.dot(z, w_ref[...], preferred_element_type=jnp.float32)
    o = jnp.maximum(o + b_ref[...], 0.0)
    if scale_out:
        o = o * d1
    outa_ref[...] = o[:, :128]
    outb_ref[...] = o[:, 128:]


def _layer(va, vb, o1a, o1b, o2a, o2b, o3a, o3b, dis, w, b, scale_out):
    h = va.shape[1]
    fin = 2 * h
    fspec = pl.BlockSpec((R, h), lambda i: (i, 0))
    return pl.pallas_call(
        functools.partial(_layer_body, scale_out),
        grid=(NB,),
        in_specs=[
            fspec, fspec, fspec, fspec, fspec, fspec, fspec, fspec,
            pl.BlockSpec((R, 16), lambda i: (i, 0)),
            pl.BlockSpec((fin, 256), lambda i: (0, 0)),
            pl.BlockSpec((1, 256), lambda i: (0, 0)),
        ],
        out_specs=[
            pl.BlockSpec((R, 128), lambda i: (i, 0)),
            pl.BlockSpec((R, 128), lambda i: (i, 0)),
        ],
        out_shape=[
            jax.ShapeDtypeStruct((NA, 128), jnp.float32),
            jax.ShapeDtypeStruct((NA, 128), jnp.float32),
        ],
    )(va, vb, o1a, o1b, o2a, o2b, o3a, o3b, dis, w, b)


def _final_body(ha_ref, hb_ref, w_ref, b_ref, out_ref):
    hfull = jnp.concatenate([ha_ref[...], hb_ref[...]], axis=1)
    o = jnp.dot(hfull, w_ref[...], preferred_element_type=jnp.float32)
    o = o + b_ref[...]
    m = jnp.max(o, axis=1, keepdims=True)
    e = jnp.exp(o - m)
    se = jnp.sum(e, axis=1, keepdims=True)
    out_ref[...] = o - m - jnp.log(se)


def _final(ha, hb, wl, bl):
    return pl.pallas_call(
        _final_body,
        grid=(NB,),
        in_specs=[
            pl.BlockSpec((R, 128), lambda i: (i, 0)),
            pl.BlockSpec((R, 128), lambda i: (i, 0)),
            pl.BlockSpec((256, 64), lambda i: (0, 0)),
            pl.BlockSpec((1, 64), lambda i: (0, 0)),
        ],
        out_specs=pl.BlockSpec((R, 64), lambda i: (i, 0)),
        out_shape=jax.ShapeDtypeStruct((N, 64), jnp.float32),
    )(ha, hb, wl, bl)


def kernel(x, edge_index, batch, W0, b0, W1, b1, W2, b2, Wl, bl):
    del batch
    src = edge_index[0]
    dst = edge_index[1]
    pad = jnp.full((EPAD - E,), PADN, jnp.int32)
    src2 = jnp.concatenate([src, pad]).reshape(NS * NCH, C)
    dst2 = jnp.concatenate([dst, pad]).reshape(NS * NCH, C)

    ones16 = jnp.ones((NA, 16), jnp.float32)
    c1, _, c2, _, c3, _ = _make_prop3(16)(ones16, ones16, src2, dst2)

    dis, v0a, v0b = _prep(x, c1[:N], c2[:N], c3[:N])

    o1a, o1b, o2a, o2b, o3a, o3b = _make_prop3(64)(v0a, v0b, src2, dst2)
    v1a, v1b = _layer(v0a, v0b, o1a, o1b, o2a, o2b, o3a, o3b, dis,
                      W0, b0.reshape(1, 256), True)

    o1a, o1b, o2a, o2b, o3a, o3b = _make_prop3(128)(v1a, v1b, src2, dst2)
    v2a, v2b = _layer(v1a, v1b, o1a, o1b, o2a, o2b, o3a, o3b, dis,
                      W1, b1.reshape(1, 256), True)

    o1a, o1b, o2a, o2b, o3a, o3b = _make_prop3(128)(v2a, v2b, src2, dst2)
    h3a, h3b = _layer(v2a, v2b, o1a, o1b, o2a, o2b, o3a, o3b, dis,
                      W2, b2.reshape(1, 256), False)

    return _final(h3a, h3b, Wl, bl.reshape(1, 64))


# speedup vs baseline: 1.2329x; 1.2264x over previous
import functools

import jax
import jax.numpy as jnp
import numpy as np
from jax import lax
from jax.experimental import pallas as pl
from jax.experimental.pallas import tpu as pltpu
from jax.experimental.pallas import tpu_sc as plsc

N = 10000
NA = 10240
PADN = NA - 8
E = 320000
L = 3
_W = [float(np.exp(-l)) for l in range(L + 1)]

NC = 2
NS = 16
LANES = 16
C = 128
NCH = 160
EPT = C * NCH
EPAD = EPT * NS
NPT = NA // NS
RZ = 32


@functools.cache
def _make_prop3(h):
    mesh = plsc.VectorSubcoreMesh(core_axis_name="c", subcore_axis_name="s")
    out_sds = jax.ShapeDtypeStruct((NA, h), jnp.float32)

    @functools.partial(
        pl.kernel,
        out_type=(out_sds,) * 6,
        mesh=mesh,
        scratch_types=[
            [pltpu.VMEM((C,), jnp.int32) for _ in range(2)],
            [pltpu.VMEM((C,), jnp.int32) for _ in range(2)],
            [pltpu.VMEM((C, h), jnp.float32) for _ in range(2)],
            pltpu.VMEM((RZ, h), jnp.float32),
            pltpu.VMEM_SHARED((NA, h), jnp.float32),
            [pltpu.SemaphoreType.DMA for _ in range(2)],
            [pltpu.SemaphoreType.DMA for _ in range(2)],
        ],
        compiler_params=pltpu.CompilerParams(use_tc_tiling_on_sc=False),
    )
    def prop3(tblA, tblB, src2, dst2, o1A, o1B, o2A, o2B, o3A, o3B,
              sib, dib, gb, zblk, acc, isem, gsem):
        c = lax.axis_index("c")
        s = lax.axis_index("s")

        per_row = h // LANES

        def zb(i, _):
            zblk[i // per_row, pl.ds((i % per_row) * LANES, LANES)] = (
                jnp.zeros((LANES,), jnp.float32))
            return 0

        lax.fori_loop(0, RZ * per_row, zb, 0)

        def idx_fetch(ch, b):
            row = s * NCH + ch
            pltpu.async_copy(src2.at[row], sib[b], isem[b])
            pltpu.async_copy(dst2.at[row], dib[b], isem[b])

        def idx_wait(ch, b):
            row = s * NCH + ch
            pltpu.make_async_copy(src2.at[row], sib[b], isem[b]).wait()
            pltpu.make_async_copy(dst2.at[row], dib[b], isem[b]).wait()

        def edge_phase(tref):
            idx_fetch(0, 0)
            idx_fetch(1, 1)
            idx_wait(0, 0)
            pltpu.async_copy(tref.at[sib[0]], gb[0], gsem[0])

            def body(j, _):
                for b in range(2):
                    i = j * 2 + b
                    nb = 1 - b

                    @pl.when(i + 1 < NCH)
                    def _():
                        idx_wait(i + 1, nb)
                        pltpu.async_copy(tref.at[sib[nb]], gb[nb], gsem[nb])

                    pltpu.make_async_copy(
                        tref.at[sib[b]], gb[b], gsem[b]).wait()
                    pltpu.sync_copy(gb[b], acc.at[dib[b]], add=True)

                    @pl.when(i + 2 < NCH)
                    def _():
                        idx_fetch(i + 2, b)
                return 0

            lax.fori_loop(0, NCH // 2, body, 0)

        def one_round(ta, tb, oa, ob):
            for k in range(NPT // RZ):
                pltpu.sync_copy(zblk, acc.at[pl.ds(s * NPT + k * RZ, RZ)])
            plsc.subcore_barrier()

            @pl.when(c == 0)
            def _():
                edge_phase(ta)

            @pl.when(c == 1)
            def _():
                edge_phase(tb)

            plsc.subcore_barrier()

            @pl.when(c == 0)
            def _():
                pltpu.sync_copy(acc.at[pl.ds(s * NPT, NPT)],
                                oa.at[pl.ds(s * NPT, NPT)])

            @pl.when(c == 1)
            def _():
                pltpu.sync_copy(acc.at[pl.ds(s * NPT, NPT)],
                                ob.at[pl.ds(s * NPT, NPT)])

            plsc.subcore_barrier()

        one_round(tblA, tblB, o1A, o1B)
        one_round(o1A, o1B, o2A, o2B)
        one_round(o2A, o2B, o3A, o3B)

    return prop3


GD = EPT // LANES
UNR = 4


@functools.cache
def _make_deg():
    mesh = plsc.VectorSubcoreMesh(core_axis_name="c", subcore_axis_name="s")
    out_sds = jax.ShapeDtypeStruct((NA,), jnp.float32)

    @functools.partial(
        pl.kernel,
        out_type=(out_sds,) * 3,
        mesh=mesh,
        scratch_types=[
            pltpu.VMEM((GD, LANES), jnp.int32),
            pltpu.VMEM((GD, LANES), jnp.int32),
            pltpu.VMEM((NA,), jnp.float32),
            pltpu.VMEM((NA,), jnp.float32),
            pltpu.VMEM((NPT,), jnp.float32),
            pltpu.VMEM((NPT,), jnp.float32),
            pltpu.VMEM_SHARED((NS, NA), jnp.float32),
            pltpu.VMEM_SHARED((NA,), jnp.float32),
        ],
        compiler_params=pltpu.CompilerParams(
            use_tc_tiling_on_sc=False, needs_layout_passes=False),
    )
    def degk(srcg, dstg, c1, c2, c3, sidx, didx, curv, accv, tstr, rstr,
             stage, sfull):
        c = lax.axis_index("c")
        s = lax.axis_index("s")
        pltpu.sync_copy(srcg.at[pl.ds(s * GD, GD)], sidx)
        pltpu.sync_copy(dstg.at[pl.ds(s * GD, GD)], didx)
        ones = jnp.ones((LANES,), jnp.float32)

        def zero_accv():
            def zb(i, _):
                accv[pl.ds(i * LANES, LANES)] = jnp.zeros(
                    (LANES,), jnp.float32)
                return 0
            lax.fori_loop(0, NA // LANES, zb, 0)

        def scatter_round(gather):
            def body(j, _):
                for u in range(UNR):
                    g = j * UNR + u
                    if gather:
                        v = plsc.load_gather(curv, [sidx[g]])
                    else:
                        v = ones
                    plsc.addupdate_scatter(accv, [didx[g]], v)
                return 0
            lax.fori_loop(0, GD // UNR, body, 0)

        def reduce_rounds(cout, readback):
            pltpu.sync_copy(accv, stage.at[s])
            plsc.subcore_barrier()
            base = s * NPT
            pltpu.sync_copy(stage.at[0, pl.ds(base, NPT)], rstr)
            for t in range(1, NS):
                pltpu.sync_copy(stage.at[t, pl.ds(base, NPT)], tstr)

                def addb(i, _):
                    rstr[pl.ds(i * LANES, LANES)] = (
                        rstr[pl.ds(i * LANES, LANES)]
                        + tstr[pl.ds(i * LANES, LANES)])
                    return 0
                lax.fori_loop(0, NPT // LANES, addb, 0)
            pltpu.sync_copy(rstr, sfull.at[pl.ds(base, NPT)])
            plsc.subcore_barrier()
            if readback:
                pltpu.sync_copy(sfull, curv)

            @pl.when(jnp.logical_and(c == 0, s == 0))
            def _():
                pltpu.sync_copy(sfull, cout)

            plsc.subcore_barrier()

        zero_accv()
        scatter_round(False)
        reduce_rounds(c1, True)
        zero_accv()
        scatter_round(True)
        reduce_rounds(c2, True)
        zero_accv()
        scatter_round(True)
        reduce_rounds(c3, False)

    return degk


R = 1000
NB = N // R


def _prep_body(x_ref, c1_ref, c2_ref, c3_ref, dis_ref, v0a_ref, v0b_ref):
    deg = (_W[0] + _W[1] * c1_ref[...] + _W[2] * c2_ref[...]
           + _W[3] * c3_ref[...])
    safe = jnp.where(deg > 0, deg, 1.0)
    dis = jnp.where(deg > 0, 1.0 / jnp.sqrt(safe), 0.0)
    dis_ref[...] = dis
    xv = x_ref[...] * dis[:, :1]
    v0a_ref[...] = xv[:, :64]
    v0b_ref[...] = xv[:, 64:]


def _prep(x, c1, c2, c3):
    return pl.pallas_call(
        _prep_body,
        grid=(NB,),
        in_specs=[
            pl.BlockSpec((R, 128), lambda i: (i, 0)),
            pl.BlockSpec((R, 16), lambda i: (i, 0)),
            pl.BlockSpec((R, 16), lambda i: (i, 0)),
            pl.BlockSpec((R, 16), lambda i: (i, 0)),
        ],
        out_specs=[
            pl.BlockSpec((R, 16), lambda i: (i, 0)),
            pl.BlockSpec((R, 64), lambda i: (i, 0)),
            pl.BlockSpec((R, 64), lambda i: (i, 0)),
        ],
        out_shape=[
            jax.ShapeDtypeStruct((N, 16), jnp.float32),
            jax.ShapeDtypeStruct((NA, 64), jnp.float32),
            jax.ShapeDtypeStruct((NA, 64), jnp.float32),
        ],
    )(x, c1, c2, c3)


def _layer_body(scale_out, va_ref, vb_ref, o1a_ref, o1b_ref, o2a_ref,
                o2b_ref, o3a_ref, o3b_ref, dis_ref, w_ref, b_ref,
                outa_ref, outb_ref):
    def full(ra, rb):
        return jnp.concatenate([ra[...], rb[...]], axis=1)

    z = (_W[0] * full(va_ref, vb_ref) + _W[1] * full(o1a_ref, o1b_ref)
         + _W[2] * full(o2a_ref, o2b_ref) + _W[3] * full(o3a_ref, o3b_ref))
    d1 = dis_ref[...][:, :1]
    z = z * d1
    o = jnp.dot(z, w_ref[...], preferred_element_type=jnp.float32)
    o = jnp.maximum(o + b_ref[...], 0.0)
    if scale_out:
        o = o * d1
    outa_ref[...] = o[:, :128]
    outb_ref[...] = o[:, 128:]


def _layer(va, vb, o1a, o1b, o2a, o2b, o3a, o3b, dis, w, b, scale_out):
    h = va.shape[1]
    fin = 2 * h
    fspec = pl.BlockSpec((R, h), lambda i: (i, 0))
    return pl.pallas_call(
        functools.partial(_layer_body, scale_out),
        grid=(NB,),
        in_specs=[
            fspec, fspec, fspec, fspec, fspec, fspec, fspec, fspec,
            pl.BlockSpec((R, 16), lambda i: (i, 0)),
            pl.BlockSpec((fin, 256), lambda i: (0, 0)),
            pl.BlockSpec((1, 256), lambda i: (0, 0)),
        ],
        out_specs=[
            pl.BlockSpec((R, 128), lambda i: (i, 0)),
            pl.BlockSpec((R, 128), lambda i: (i, 0)),
        ],
        out_shape=[
            jax.ShapeDtypeStruct((NA, 128), jnp.float32),
            jax.ShapeDtypeStruct((NA, 128), jnp.float32),
        ],
    )(va, vb, o1a, o1b, o2a, o2b, o3a, o3b, dis, w, b)


def _final_body(ha_ref, hb_ref, w_ref, b_ref, out_ref):
    hfull = jnp.concatenate([ha_ref[...], hb_ref[...]], axis=1)
    o = jnp.dot(hfull, w_ref[...], preferred_element_type=jnp.float32)
    o = o + b_ref[...]
    m = jnp.max(o, axis=1, keepdims=True)
    e = jnp.exp(o - m)
    se = jnp.sum(e, axis=1, keepdims=True)
    out_ref[...] = o - m - jnp.log(se)


def _final(ha, hb, wl, bl):
    return pl.pallas_call(
        _final_body,
        grid=(NB,),
        in_specs=[
            pl.BlockSpec((R, 128), lambda i: (i, 0)),
            pl.BlockSpec((R, 128), lambda i: (i, 0)),
            pl.BlockSpec((256, 64), lambda i: (0, 0)),
            pl.BlockSpec((1, 64), lambda i: (0, 0)),
        ],
        out_specs=pl.BlockSpec((R, 64), lambda i: (i, 0)),
        out_shape=jax.ShapeDtypeStruct((N, 64), jnp.float32),
    )(ha, hb, wl, bl)


def kernel(x, edge_index, batch, W0, b0, W1, b1, W2, b2, Wl, bl):
    del batch
    src = edge_index[0]
    dst = edge_index[1]
    pad = jnp.full((EPAD - E,), PADN, jnp.int32)
    src2 = jnp.concatenate([src, pad]).reshape(NS * NCH, C)
    dst2 = jnp.concatenate([dst, pad]).reshape(NS * NCH, C)

    srcg = src2.reshape(NS * GD, LANES)
    dstg = dst2.reshape(NS * GD, LANES)
    c1, c2, c3 = _make_deg()(srcg, dstg)

    dis, v0a, v0b = _prep(
        x,
        jnp.broadcast_to(c1[:N, None], (N, 16)),
        jnp.broadcast_to(c2[:N, None], (N, 16)),
        jnp.broadcast_to(c3[:N, None], (N, 16)))

    o1a, o1b, o2a, o2b, o3a, o3b = _make_prop3(64)(v0a, v0b, src2, dst2)
    v1a, v1b = _layer(v0a, v0b, o1a, o1b, o2a, o2b, o3a, o3b, dis,
                      W0, b0.reshape(1, 256), True)

    o1a, o1b, o2a, o2b, o3a, o3b = _make_prop3(128)(v1a, v1b, src2, dst2)
    v2a, v2b = _layer(v1a, v1b, o1a, o1b, o2a, o2b, o3a, o3b, dis,
                      W1, b1.reshape(1, 256), True)

    o1a, o1b, o2a, o2b, o3a, o3b = _make_prop3(128)(v2a, v2b, src2, dst2)
    h3a, h3b = _layer(v2a, v2b, o1a, o1b, o2a, o2b, o3a, o3b, dis,
                      W2, b2.reshape(1, 256), False)

    return _final(h3a, h3b, Wl, bl.reshape(1, 64))
